# baseline (device time: 22860 ns/iter reference)
import jax
import jax.numpy as jnp
from jax import lax
from jax.experimental import pallas as pl
from jax.experimental.pallas import tpu as pltpu

N_DEV = 4
B_LOC = 2
SQ = 128
SKV = 128
HQ = 16
H_LOC = 4
H_HALF = 2
DH = 64
D_MODEL = 512
D_GRP = H_LOC * DH
D_HALF = D_GRP // 2


def kernel(x, Wq, K_ext, V_ext, Wo):
    n_b_glob = K_ext.shape[0]
    k2 = K_ext.reshape(n_b_glob, SQ, HQ * DH)
    v2 = V_ext.reshape(n_b_glob, SQ, HQ * DH)
    x2 = x.reshape(B_LOC * SQ, D_MODEL)

    def body(x_ref, wq_ref, k_hbm, v_hbm, wo_ref, out_ref,
             x_bf, k_f32, v_f32, wqR, wqL, woR, woL, local_sems, ssem, rsem):
        my_pos = lax.axis_index("i")
        left = lax.rem(my_pos + N_DEV - 1, N_DEV)
        right = lax.rem(my_pos + 1, N_DEV)

        k_dma = pltpu.make_async_copy(
            k_hbm.at[pl.ds(my_pos * B_LOC, B_LOC)], k_f32, local_sems.at[0])
        v_dma = pltpu.make_async_copy(
            v_hbm.at[pl.ds(my_pos * B_LOC, B_LOC)], v_f32, local_sems.at[1])
        k_dma.start()
        v_dma.start()

        barrier_sem = pltpu.get_barrier_semaphore()
        for nbr in (left, right):
            pl.semaphore_signal(
                barrier_sem, inc=1,
                device_id=(nbr,), device_id_type=pl.DeviceIdType.MESH,
            )
        pl.semaphore_wait(barrier_sem, 2)

        qb = lax.broadcasted_iota(jnp.int32, (SQ, SKV), 0) // 64
        kb = lax.broadcasted_iota(jnp.int32, (SQ, SKV), 1) // 64
        mask = (qb == kb) | (kb == 0) | (lax.rem(qb + kb, 3) == 0)
        neg = jnp.float32(-1e9)

        descs = [
            (wqR, 0, 1, right), (woR, 0, 1, right),
            (wqL, 0, 3, left),  (woL, 0, 3, left),
            (wqL, 0, 1, right), (woL, 0, 1, right),
            (wqR, 0, 3, left),  (woR, 0, 3, left),
            (wqR, 1, 2, right), (woR, 1, 2, right),
            (wqL, 3, 2, left),  (woL, 3, 2, left),
        ]
        rdmas = [
            pltpu.make_async_remote_copy(
                src_ref=buf.at[s_slot], dst_ref=buf.at[d_slot],
                send_sem=ssem.at[i], recv_sem=rsem.at[i],
                device_id=(dst,), device_id_type=pl.DeviceIdType.MESH,
            )
            for i, (buf, s_slot, d_slot, dst) in enumerate(descs)
        ]

        def compute_half(slot, origin, half, first=False):
            wq_h = (wqR if half == 0 else wqL)[slot]
            wo_h = (woR if half == 0 else woL)[slot]
            c0 = origin * D_GRP + half * D_HALF
            q_all = lax.dot_general(
                x_bf[:, :], wq_h, (((1,), (0,)), ((), ())),
                preferred_element_type=jnp.float32,
            ).astype(jnp.bfloat16)
            ctx_rows = []
            for b in range(B_LOC):
                k_pair = k_f32[b, :, pl.ds(c0, D_HALF)].astype(jnp.bfloat16)
                v_pair = v_f32[b, :, pl.ds(c0, D_HALF)].astype(jnp.bfloat16)
                ctx_parts = []
                for hh in range(H_HALF):
                    q = q_all[b * SQ:(b + 1) * SQ, hh * DH:(hh + 1) * DH]
                    scores = lax.dot_general(
                        q, k_pair[:, hh * DH:(hh + 1) * DH],
                        (((1,), (1,)), ((), ())),
                        preferred_element_type=jnp.float32,
                    ) * 0.125
                    w = jnp.exp(jnp.where(mask, scores, neg))
                    r = 1.0 / jnp.sum(w, axis=-1, keepdims=True)
                    ctx = lax.dot_general(
                        w.astype(jnp.bfloat16), v_pair[:, hh * DH:(hh + 1) * DH],
                        (((1,), (0,)), ((), ())),
                        preferred_element_type=jnp.float32,
                    ) * r
                    ctx_parts.append(ctx.astype(jnp.bfloat16))
                ctx_rows.append(jnp.concatenate(ctx_parts, axis=1))
            ctx_all = jnp.concatenate(ctx_rows, axis=0)
            partial = lax.dot_general(
                ctx_all, wo_h, (((1,), (0,)), ((), ())),
                preferred_element_type=jnp.float32,
            )
            if first:
                out_ref[:, :] = partial
            else:
                out_ref[:, :] = out_ref[:, :] + partial

        x_bf[:, :] = x_ref[:, :].astype(jnp.bfloat16)
        wqR[0] = wq_ref[:, 0:D_HALF].astype(jnp.bfloat16)
        wqL[0] = wq_ref[:, D_HALF:D_GRP].astype(jnp.bfloat16)
        woR[0] = wo_ref[0:D_HALF, :].astype(jnp.bfloat16)
        woL[0] = wo_ref[D_HALF:D_GRP, :].astype(jnp.bfloat16)

        for i in (0, 2, 1, 3, 4, 6, 5, 7):
            rdmas[i].start()

        k_dma.wait()
        v_dma.wait()
        compute_half(0, my_pos, 0, first=True)
        compute_half(0, my_pos, 1)

        rdmas[0].wait_recv()
        rdmas[1].wait_recv()
        rdmas[8].start()
        rdmas[9].start()
        rdmas[2].wait_recv()
        rdmas[3].wait_recv()
        rdmas[10].start()
        rdmas[11].start()

        compute_half(1, left, 0)
        compute_half(3, right, 1)
        rdmas[4].wait_recv()
        rdmas[5].wait_recv()
        compute_half(1, left, 1)
        rdmas[6].wait_recv()
        rdmas[7].wait_recv()
        compute_half(3, right, 0)
        diag = lax.rem(my_pos + 2, N_DEV)
        rdmas[8].wait_recv()
        rdmas[9].wait_recv()
        compute_half(2, diag, 0)
        rdmas[10].wait_recv()
        rdmas[11].wait_recv()
        compute_half(2, diag, 1)

        for rdma in rdmas:
            rdma.wait_send()

    out = pl.pallas_call(
        body,
        out_shape=jax.ShapeDtypeStruct((B_LOC * SQ, D_MODEL), jnp.float32),
        in_specs=[
            pl.BlockSpec(memory_space=pltpu.VMEM),
            pl.BlockSpec(memory_space=pltpu.VMEM),
            pl.BlockSpec(memory_space=pltpu.HBM),
            pl.BlockSpec(memory_space=pltpu.HBM),
            pl.BlockSpec(memory_space=pltpu.VMEM),
        ],
        out_specs=pl.BlockSpec(memory_space=pltpu.VMEM),
        scratch_shapes=[
            pltpu.VMEM((B_LOC * SQ, D_MODEL), jnp.bfloat16),
            pltpu.VMEM((B_LOC, SQ, HQ * DH), jnp.float32),
            pltpu.VMEM((B_LOC, SQ, HQ * DH), jnp.float32),
            pltpu.VMEM((N_DEV, D_MODEL, D_HALF), jnp.bfloat16),
            pltpu.VMEM((N_DEV, D_MODEL, D_HALF), jnp.bfloat16),
            pltpu.VMEM((N_DEV, D_HALF, D_MODEL), jnp.bfloat16),
            pltpu.VMEM((N_DEV, D_HALF, D_MODEL), jnp.bfloat16),
            pltpu.SemaphoreType.DMA((2,)),
            pltpu.SemaphoreType.DMA((12,)),
            pltpu.SemaphoreType.DMA((12,)),
        ],
        compiler_params=pltpu.CompilerParams(collective_id=0),
    )(x2, Wq, k2, v2, Wo)
    return out.reshape(B_LOC, SQ, D_MODEL)
